# baseline (speedup 1.0000x reference)
import functools

import jax
import jax.numpy as jnp
from jax import lax
from jax.experimental import pallas as pl
from jax.experimental.pallas import tpu as pltpu
from jax.experimental.pallas import tpu_sc as plsc

B = 16384
D = 64
DP = 128
NC = 2
NS = 16
NW = NC * NS
BPW = B // NW
G = 128
NG = BPW // G


def _gmf_body(uid_hbm, iid_hbm, ut_hbm, it_hbm, w_hbm, b_hbm, out_hbm,
              idx_u, idx_i, rows_u, rows_i, w_v, b_v, out_v, sem_u, sem_i):
    wid = lax.axis_index("s") * NC + lax.axis_index("c")
    base = wid * BPW

    pltpu.sync_copy(uid_hbm.at[pl.ds(base, BPW)], idx_u)
    pltpu.sync_copy(iid_hbm.at[pl.ds(base, BPW)], idx_i)
    pltpu.sync_copy(w_hbm, w_v)
    pltpu.sync_copy(b_hbm, b_v)

    w0 = w_v[pl.ds(0, 16)]
    w1 = w_v[pl.ds(16, 16)]
    w2 = w_v[pl.ds(32, 16)]
    w3 = w_v[pl.ds(48, 16)]
    bvec = b_v[...]
    lane = lax.iota(jnp.int32, 16)

    for g in range(NG):
        du = pltpu.async_copy(
            ut_hbm.at[idx_u.at[pl.ds(g * G, G)]], rows_u, sem_u)
        di = pltpu.async_copy(
            it_hbm.at[idx_i.at[pl.ds(g * G, G)]], rows_i, sem_i)
        du.wait()
        di.wait()

        def group(q, carry, g=g):
            vec = jnp.zeros((16,), jnp.float32)
            for k in range(16):
                r = q * 16 + k
                acc = rows_u[r, pl.ds(0, 16)] * rows_i[r, pl.ds(0, 16)] * w0
                acc += rows_u[r, pl.ds(16, 16)] * rows_i[r, pl.ds(16, 16)] * w1
                acc += rows_u[r, pl.ds(32, 16)] * rows_i[r, pl.ds(32, 16)] * w2
                acc += rows_u[r, pl.ds(48, 16)] * rows_i[r, pl.ds(48, 16)] * w3
                vec = jnp.where(lane == k, jnp.sum(acc), vec)
            out_v[pl.ds(g * G + q * 16, 16)] = vec + bvec
            return carry

        lax.fori_loop(0, G // 16, group, 0)

    pltpu.sync_copy(out_v, out_hbm.at[pl.ds(base, BPW)])


@jax.jit
def kernel(userID, itemID, user_table, item_table, W, b):
    ut_p = jnp.pad(user_table, ((0, 0), (0, DP - D)))
    it_p = jnp.pad(item_table, ((0, 0), (0, DP - D)))
    w1d = W.reshape(D)
    b16 = jnp.broadcast_to(b.astype(jnp.float32), (16,))

    mesh = plsc.VectorSubcoreMesh(core_axis_name="c", subcore_axis_name="s")
    f = pl.kernel(
        _gmf_body,
        mesh=mesh,
        compiler_params=pltpu.CompilerParams(needs_layout_passes=False),
        out_type=jax.ShapeDtypeStruct((B,), jnp.float32),
        scratch_types=[
            pltpu.VMEM((BPW,), jnp.int32),
            pltpu.VMEM((BPW,), jnp.int32),
            pltpu.VMEM((G, DP), jnp.float32),
            pltpu.VMEM((G, DP), jnp.float32),
            pltpu.VMEM((D,), jnp.float32),
            pltpu.VMEM((16,), jnp.float32),
            pltpu.VMEM((BPW,), jnp.float32),
            pltpu.SemaphoreType.DMA,
            pltpu.SemaphoreType.DMA,
        ],
    )
    return f(userID, itemID, ut_p, it_p, w1d, b16)
